# baseline (speedup 1.0000x reference)
import functools

import jax
import jax.numpy as jnp
from jax import lax
from jax.experimental import pallas as pl
from jax.experimental.pallas import tpu as pltpu
from jax.experimental.pallas import tpu_sc as plsc

N_NODES = 100000
EMB_DIM = 128
PRIOR_DIM = 64
OUT_DIM = PRIOR_DIM + EMB_DIM
CHUNK = 128
NUM_FULL = N_NODES // CHUNK
REM = N_NODES - NUM_FULL * CHUNK
NUM_CHUNKS_PAD = NUM_FULL + 1
NW = 32
NMAX = (NUM_FULL + NW - 1) // NW
LAST_FULL_W = (NUM_FULL - 1) % NW


def _build_kernel():
    mesh = plsc.VectorSubcoreMesh(core_axis_name="c", subcore_axis_name="s")

    @functools.partial(
        pl.kernel,
        mesh=mesh,
        compiler_params=pltpu.CompilerParams(use_tc_tiling_on_sc=False),
        out_type=jax.ShapeDtypeStruct((N_NODES, OUT_DIM), jnp.float32),
        scratch_types=[
            pltpu.VMEM((2, CHUNK), jnp.int32),
            pltpu.VMEM((2, CHUNK, EMB_DIM), jnp.float32),
            pltpu.SemaphoreType.DMA,
            pltpu.SemaphoreType.DMA,
            pltpu.SemaphoreType.DMA,
            pltpu.SemaphoreType.DMA,
            pltpu.SemaphoreType.DMA,
            pltpu.SemaphoreType.DMA,
            pltpu.SemaphoreType.DMA,
        ],
    )
    def k(idx_hbm, inv_hbm, tab_hbm, out_hbm, idx_v, emb_v,
          gs0, gs1, is0, is1, ws0, ws1, invs):
        gsem = (gs0, gs1)
        isem = (is0, is1)
        wsem = (ws0, ws1)
        cid = lax.axis_index("c")
        sid = lax.axis_index("s")
        wid = sid * 2 + cid
        n = jnp.where(wid <= LAST_FULL_W, NMAX, NMAX - 1)

        def rows(c):
            return pl.ds(c * CHUNK, CHUNK)

        def issue_gather(slot):
            pltpu.async_copy(tab_hbm.at[idx_v.at[slot]], emb_v.at[slot],
                             gsem[slot])

        def wait_gather(slot):
            pltpu.make_async_copy(tab_hbm.at[idx_v.at[slot]], emb_v.at[slot],
                                  gsem[slot]).wait()

        def issue_idx(t, slot):
            pltpu.async_copy(idx_hbm.at[wid + NW * t], idx_v.at[slot],
                             isem[slot])

        def wait_idx(slot):
            pltpu.make_async_copy(idx_hbm.at[0], idx_v.at[slot],
                                  isem[slot]).wait()

        def issue_writes(t, slot):
            c = wid + NW * t
            pltpu.async_copy(emb_v.at[slot],
                             out_hbm.at[rows(c), pl.ds(PRIOR_DIM, EMB_DIM)],
                             wsem[slot])
            pltpu.async_copy(inv_hbm.at[rows(c), :],
                             out_hbm.at[rows(c), pl.ds(0, PRIOR_DIM)],
                             invs)

        def wait_write(slot):
            pltpu.make_async_copy(
                emb_v.at[slot],
                out_hbm.at[rows(0), pl.ds(PRIOR_DIM, EMB_DIM)],
                wsem[slot]).wait()

        def wait_inv():
            pltpu.make_async_copy(
                inv_hbm.at[rows(0), :],
                out_hbm.at[rows(0), pl.ds(0, PRIOR_DIM)],
                invs).wait()

        pltpu.sync_copy(idx_hbm.at[wid], idx_v.at[0])
        issue_gather(0)
        issue_idx(1, 1)

        def half(cur, t_cur):
            nxt = 1 - cur
            t_nxt = t_cur + 1

            @pl.when(t_nxt < n)
            def _():
                wait_idx(nxt)

                @pl.when(t_nxt >= 2)
                def _():
                    wait_write(nxt)

                issue_gather(nxt)

            @pl.when(t_cur < n)
            def _():
                wait_gather(cur)

                @pl.when(t_cur + 2 < n)
                def _():
                    issue_idx(t_cur + 2, cur)

                issue_writes(t_cur, cur)

        def body(p, carry):
            half(0, 2 * p)
            half(1, 2 * p + 1)
            return carry

        lax.fori_loop(0, (NMAX + 1) // 2, body, 0)

        wait_write(0)
        wait_write(1)
        lax.fori_loop(0, n, lambda t, c: (wait_inv(), c)[1], 0)

        @pl.when(wid == NW - 1)
        def _tail():
            base = NUM_FULL * CHUNK
            pltpu.sync_copy(idx_hbm.at[NUM_FULL], idx_v.at[0])
            issue_gather(0)
            wait_gather(0)
            pltpu.sync_copy(emb_v.at[0, pl.ds(0, REM)],
                            out_hbm.at[pl.ds(base, REM),
                                       pl.ds(PRIOR_DIM, EMB_DIM)])
            pltpu.sync_copy(inv_hbm.at[pl.ds(base, REM), :],
                            out_hbm.at[pl.ds(base, REM), pl.ds(0, PRIOR_DIM)])

    return k


_KERNEL = _build_kernel()


def kernel(feature, invariant_node_features, table):
    feat = feature.astype(jnp.int32)
    pad = NUM_CHUNKS_PAD * CHUNK - N_NODES
    feat2d = jnp.pad(feat, (0, pad)).reshape(NUM_CHUNKS_PAD, CHUNK)
    return _KERNEL(feat2d, invariant_node_features, table)


# speedup vs baseline: 5.5235x; 5.5235x over previous
import functools

import jax
import jax.numpy as jnp
from jax import lax
from jax.experimental import pallas as pl
from jax.experimental.pallas import tpu as pltpu
from jax.experimental.pallas import tpu_sc as plsc

N_NODES = 100000
EMB_DIM = 128
PRIOR_DIM = 64
OUT_DIM = PRIOR_DIM + EMB_DIM
CHUNK = 128
NUM_FULL = N_NODES // CHUNK
REM = N_NODES - NUM_FULL * CHUNK
N_PAD = (NUM_FULL + 1) * CHUNK
NW = 32
NMAX = (NUM_FULL + NW - 1) // NW
LAST_FULL_W = (NUM_FULL - 1) % NW


def _build_kernel():
    mesh = plsc.VectorSubcoreMesh(core_axis_name="c", subcore_axis_name="s")

    @functools.partial(
        pl.kernel,
        mesh=mesh,
        out_type=jax.ShapeDtypeStruct((N_NODES, OUT_DIM), jnp.float32),
        scratch_types=[
            pltpu.VMEM((CHUNK,), jnp.int32),
            pltpu.VMEM((CHUNK,), jnp.int32),
            pltpu.VMEM((CHUNK, OUT_DIM), jnp.float32),
            pltpu.VMEM((CHUNK, OUT_DIM), jnp.float32),
            pltpu.VMEM((CHUNK, PRIOR_DIM), jnp.float32),
            pltpu.VMEM((CHUNK, PRIOR_DIM), jnp.float32),
            pltpu.SemaphoreType.DMA,
            pltpu.SemaphoreType.DMA,
            pltpu.SemaphoreType.DMA,
            pltpu.SemaphoreType.DMA,
            pltpu.SemaphoreType.DMA,
            pltpu.SemaphoreType.DMA,
            pltpu.SemaphoreType.DMA,
            pltpu.SemaphoreType.DMA,
        ],
    )
    def k(feat_hbm, inv_hbm, tab_hbm, out_hbm,
          idx0, idx1, buf0, buf1, pri0, pri1,
          gs0, gs1, vs0, vs1, is0, is1, ws0, ws1):
        idx = (idx0, idx1)
        buf = (buf0, buf1)
        pri = (pri0, pri1)
        gsem = (gs0, gs1)
        vsem = (vs0, vs1)
        isem = (is0, is1)
        wsem = (ws0, ws1)
        cid = lax.axis_index("c")
        sid = lax.axis_index("s")
        wid = sid * 2 + cid
        n = jnp.where(wid <= LAST_FULL_W, NMAX, NMAX - 1)

        def rows(c):
            return pl.ds(c * CHUNK, CHUNK)

        def issue_gather(s):
            pltpu.async_copy(tab_hbm.at[idx[s]],
                             buf[s].at[:, pl.ds(0, EMB_DIM)], gsem[s])

        def wait_gather(s):
            pltpu.make_async_copy(tab_hbm.at[idx[s]],
                                  buf[s].at[:, pl.ds(0, EMB_DIM)],
                                  gsem[s]).wait()

        def issue_pri(t, s):
            pltpu.async_copy(inv_hbm.at[rows(wid + NW * t), :], pri[s],
                             vsem[s])

        def wait_pri(s):
            pltpu.make_async_copy(inv_hbm.at[rows(0), :], pri[s],
                                  vsem[s]).wait()

        def issue_idx(t, s):
            pltpu.async_copy(feat_hbm.at[pl.ds((wid + NW * t) * CHUNK, CHUNK)],
                             idx[s], isem[s])

        def wait_idx(s):
            pltpu.make_async_copy(feat_hbm.at[pl.ds(0, CHUNK)], idx[s],
                                  isem[s]).wait()

        def issue_write(t, s):
            pltpu.async_copy(buf[s], out_hbm.at[rows(wid + NW * t), :],
                             wsem[s])

        def wait_write(s):
            pltpu.make_async_copy(buf[s], out_hbm.at[rows(0), :],
                                  wsem[s]).wait()

        def merge(s, nrows):
            b, p = buf[s], pri[s]

            def row(r, carry):
                for g in range(4):
                    b[r, pl.ds(EMB_DIM + g * 16, 16)] = \
                        b[r, pl.ds(PRIOR_DIM + g * 16, 16)]
                for g in range(4):
                    b[r, pl.ds(PRIOR_DIM + g * 16, 16)] = \
                        b[r, pl.ds(g * 16, 16)]
                for g in range(4):
                    b[r, pl.ds(g * 16, 16)] = p[r, pl.ds(g * 16, 16)]
                return carry

            lax.fori_loop(0, nrows, row, 0, unroll=2)

        pltpu.sync_copy(feat_hbm.at[pl.ds(wid * CHUNK, CHUNK)], idx[0])
        issue_gather(0)
        issue_pri(0, 0)
        issue_idx(1, 1)

        def half(cur, t_cur):
            nxt = 1 - cur
            t_nxt = t_cur + 1

            @pl.when(t_nxt < n)
            def _():
                wait_idx(nxt)

                @pl.when(t_nxt >= 2)
                def _():
                    wait_write(nxt)

                issue_gather(nxt)
                issue_pri(t_nxt, nxt)

            @pl.when(t_cur < n)
            def _():
                wait_gather(cur)
                wait_pri(cur)

                @pl.when(t_cur + 2 < n)
                def _():
                    issue_idx(t_cur + 2, cur)

                merge(cur, CHUNK)
                issue_write(t_cur, cur)

        def body(p, carry):
            half(0, 2 * p)
            half(1, 2 * p + 1)
            return carry

        lax.fori_loop(0, (NMAX + 1) // 2, body, 0)

        wait_write(0)
        wait_write(1)

        @pl.when(wid == NW - 1)
        def _tail():
            base = NUM_FULL * CHUNK
            pltpu.sync_copy(feat_hbm.at[pl.ds(base, CHUNK)], idx[0])
            issue_gather(0)
            pltpu.sync_copy(inv_hbm.at[pl.ds(base, REM), :],
                            pri[0].at[pl.ds(0, REM), :])
            wait_gather(0)
            merge(0, REM)
            pltpu.sync_copy(buf[0].at[pl.ds(0, REM), :],
                            out_hbm.at[pl.ds(base, REM), :])

    return k


_KERNEL = _build_kernel()


def kernel(feature, invariant_node_features, table):
    feat = feature.astype(jnp.int32)
    feat_pad = jnp.pad(feat, (0, N_PAD - N_NODES))
    return _KERNEL(feat_pad, invariant_node_features, table)
